# baseline (speedup 1.0000x reference)
import jax
import jax.numpy as jnp
import numpy as np
from jax.experimental import pallas as pl
from jax.experimental.pallas import tpu as pltpu

_B = 128
_V = 100000

_BVA = 20480
_NBA = pl.cdiv(_V, _BVA)
_BVO = 25088
_NBO = pl.cdiv(_V, _BVO)


def _make_gumbel():
    n = _B * _V
    idx = np.arange(n, dtype=np.uint64)
    x0 = (idx >> np.uint64(32)).astype(np.uint32)
    x1 = (idx & np.uint64(0xFFFFFFFF)).astype(np.uint32)
    k0, k1 = np.uint32(0), np.uint32(1)
    ks2 = np.uint32(k0 ^ k1 ^ np.uint32(0x1BD11BDA))
    ks = [k0, k1, ks2]
    x0 = (x0 + k0).astype(np.uint32)
    x1 = (x1 + k1).astype(np.uint32)
    rot1 = (13, 15, 26, 6)
    rot2 = (17, 29, 16, 24)

    def rotl(v, d):
        return ((v << np.uint32(d)) | (v >> np.uint32(32 - d))).astype(np.uint32)

    for i in range(5):
        for r in (rot1 if i % 2 == 0 else rot2):
            x0 = (x0 + x1).astype(np.uint32)
            x1 = rotl(x1, r)
            x1 = (x1 ^ x0).astype(np.uint32)
        x0 = (x0 + ks[(i + 1) % 3]).astype(np.uint32)
        x1 = (x1 + ks[(i + 2) % 3] + np.uint32(i + 1)).astype(np.uint32)
    bits = (x0 ^ x1).astype(np.uint32)
    m = (bits >> np.uint32(9)).astype(np.uint32)
    b0 = (m & np.uint32(0xFF)).astype(np.int8).reshape(_B, _V)
    b1 = ((m >> np.uint32(8)) & np.uint32(0xFF)).astype(np.int8).reshape(_B, _V)
    b2 = ((m >> np.uint32(16)) & np.uint32(0x7F)).astype(np.int8).reshape(_B, _V)
    return b0, b1, b2


_GB0, _GB1, _GB2 = _make_gumbel()


def _argmax_body(x_ref, b0_ref, b1_ref, b2_ref, idx_ref, vmax_ref):
    j = pl.program_id(0)
    m = ((b0_ref[...].astype(jnp.int32) & 0xFF)
         | ((b1_ref[...].astype(jnp.int32) & 0xFF) << 8)
         | (b2_ref[...].astype(jnp.int32) << 16))
    f = jax.lax.bitcast_convert_type(m | jnp.int32(0x3F800000), jnp.float32)
    u = f - jnp.float32(1.0)
    eps = jnp.float32(1e-10)
    g = -jnp.log(eps - jnp.log(u + eps))
    x = x_ref[...] + g
    col = jax.lax.broadcasted_iota(jnp.int32, (_B, _BVA), 1) + j * _BVA
    x = jnp.where(col < _V, x, -jnp.inf)
    bm = jnp.max(x, axis=1, keepdims=True)
    bidx = jnp.min(jnp.where(x == bm, col, jnp.int32(2**31 - 1)),
                   axis=1, keepdims=True)

    @pl.when(j == 0)
    def _():
        vmax_ref[...] = bm
        idx_ref[...] = bidx

    @pl.when(j > 0)
    def _():
        upd = bm > vmax_ref[...]
        vmax_ref[...] = jnp.where(upd, bm, vmax_ref[...])
        idx_ref[...] = jnp.where(upd, bidx, idx_ref[...])


def _onehot_body(idx_ref, o_ref):
    j = pl.program_id(0)
    col = jax.lax.broadcasted_iota(jnp.int32, (_B, _BVO), 1) + j * _BVO
    o_ref[...] = (col == idx_ref[...]).astype(jnp.float32)


@jax.jit
def kernel(logits):
    idx = pl.pallas_call(
        _argmax_body,
        grid=(_NBA,),
        in_specs=[pl.BlockSpec((_B, _BVA), lambda j: (0, j)),
                  pl.BlockSpec((_B, _BVA), lambda j: (0, j)),
                  pl.BlockSpec((_B, _BVA), lambda j: (0, j)),
                  pl.BlockSpec((_B, _BVA), lambda j: (0, j))],
        out_specs=pl.BlockSpec((_B, 1), lambda j: (0, 0)),
        out_shape=jax.ShapeDtypeStruct((_B, 1), jnp.int32),
        scratch_shapes=[pltpu.VMEM((_B, 1), jnp.float32)],
    )(logits, _GB0, _GB1, _GB2)
    out = pl.pallas_call(
        _onehot_body,
        grid=(_NBO,),
        in_specs=[pl.BlockSpec((_B, 1), lambda j: (0, 0))],
        out_specs=pl.BlockSpec((_B, _BVO), lambda j: (0, j)),
        out_shape=jax.ShapeDtypeStruct((_B, _V), jnp.float32),
    )(idx)
    return out


# speedup vs baseline: 1.0867x; 1.0867x over previous
import jax
import jax.numpy as jnp
import numpy as np
from jax.experimental import pallas as pl
from jax.experimental.pallas import tpu as pltpu

_B = 128
_V = 100000

_BVA = 20480
_NBA = pl.cdiv(_V, _BVA)
_BVO = 25088
_NBO = pl.cdiv(_V, _BVO)


def _make_gumbel():
    n = _B * _V
    idx = np.arange(n, dtype=np.uint64)
    x0 = (idx >> np.uint64(32)).astype(np.uint32)
    x1 = (idx & np.uint64(0xFFFFFFFF)).astype(np.uint32)
    k0, k1 = np.uint32(0), np.uint32(1)
    ks2 = np.uint32(k0 ^ k1 ^ np.uint32(0x1BD11BDA))
    ks = [k0, k1, ks2]
    x0 = (x0 + k0).astype(np.uint32)
    x1 = (x1 + k1).astype(np.uint32)
    rot1 = (13, 15, 26, 6)
    rot2 = (17, 29, 16, 24)

    def rotl(v, d):
        return ((v << np.uint32(d)) | (v >> np.uint32(32 - d))).astype(np.uint32)

    for i in range(5):
        for r in (rot1 if i % 2 == 0 else rot2):
            x0 = (x0 + x1).astype(np.uint32)
            x1 = rotl(x1, r)
            x1 = (x1 ^ x0).astype(np.uint32)
        x0 = (x0 + ks[(i + 1) % 3]).astype(np.uint32)
        x1 = (x1 + ks[(i + 2) % 3] + np.uint32(i + 1)).astype(np.uint32)
    bits = (x0 ^ x1).astype(np.uint32)
    m = (bits >> np.uint32(9)).astype(np.uint32)
    lo = (m & np.uint32(0xFFFF)).astype(np.uint16).view(np.int16).reshape(_B, _V)
    hi = (m >> np.uint32(16)).astype(np.int8).reshape(_B, _V)
    return lo, hi


_GLO, _GHI = _make_gumbel()


def _argmax_body(x_ref, lo_ref, hi_ref, idx_ref, vmax_ref):
    j = pl.program_id(0)
    m = ((lo_ref[...].astype(jnp.int32) & 0xFFFF)
         | (hi_ref[...].astype(jnp.int32) << 16))
    f = jax.lax.bitcast_convert_type(m | jnp.int32(0x3F800000), jnp.float32)
    u = f - jnp.float32(1.0)
    eps = jnp.float32(1e-10)
    g = -jnp.log(eps - jnp.log(u + eps))
    x = x_ref[...] + g
    col = jax.lax.broadcasted_iota(jnp.int32, (_B, _BVA), 1) + j * _BVA
    x = jnp.where(col < _V, x, -jnp.inf)
    bm = jnp.max(x, axis=1, keepdims=True)
    bidx = jnp.min(jnp.where(x == bm, col, jnp.int32(2**31 - 1)),
                   axis=1, keepdims=True)

    @pl.when(j == 0)
    def _():
        vmax_ref[...] = bm
        idx_ref[...] = bidx

    @pl.when(j > 0)
    def _():
        upd = bm > vmax_ref[...]
        vmax_ref[...] = jnp.where(upd, bm, vmax_ref[...])
        idx_ref[...] = jnp.where(upd, bidx, idx_ref[...])


def _onehot_body(idx_ref, o_ref):
    j = pl.program_id(0)
    col = jax.lax.broadcasted_iota(jnp.int32, (_B, _BVO), 1) + j * _BVO
    o_ref[...] = (col == idx_ref[...]).astype(jnp.float32)


@jax.jit
def kernel(logits):
    idx = pl.pallas_call(
        _argmax_body,
        grid=(_NBA,),
        in_specs=[pl.BlockSpec((_B, _BVA), lambda j: (0, j)),
                  pl.BlockSpec((_B, _BVA), lambda j: (0, j)),
                  pl.BlockSpec((_B, _BVA), lambda j: (0, j))],
        out_specs=pl.BlockSpec((_B, 1), lambda j: (0, 0)),
        out_shape=jax.ShapeDtypeStruct((_B, 1), jnp.int32),
        scratch_shapes=[pltpu.VMEM((_B, 1), jnp.float32)],
    )(logits, _GLO, _GHI)
    out = pl.pallas_call(
        _onehot_body,
        grid=(_NBO,),
        in_specs=[pl.BlockSpec((_B, 1), lambda j: (0, 0))],
        out_specs=pl.BlockSpec((_B, _BVO), lambda j: (0, j)),
        out_shape=jax.ShapeDtypeStruct((_B, _V), jnp.float32),
    )(idx)
    return out


# speedup vs baseline: 1.1965x; 1.1011x over previous
import jax
import jax.numpy as jnp
import numpy as np
from jax.experimental import pallas as pl
from jax.experimental.pallas import tpu as pltpu

_B = 128
_V = 100000

_BVA = 20480
_NBA = pl.cdiv(_V, _BVA)
_BVO = 25088
_NBO = pl.cdiv(_V, _BVO)


def _make_gumbel():
    n = _B * _V
    idx = np.arange(n, dtype=np.uint64)
    x0 = (idx >> np.uint64(32)).astype(np.uint32)
    x1 = (idx & np.uint64(0xFFFFFFFF)).astype(np.uint32)
    k0, k1 = np.uint32(0), np.uint32(1)
    ks2 = np.uint32(k0 ^ k1 ^ np.uint32(0x1BD11BDA))
    ks = [k0, k1, ks2]
    x0 = (x0 + k0).astype(np.uint32)
    x1 = (x1 + k1).astype(np.uint32)
    rot1 = (13, 15, 26, 6)
    rot2 = (17, 29, 16, 24)

    def rotl(v, d):
        return ((v << np.uint32(d)) | (v >> np.uint32(32 - d))).astype(np.uint32)

    for i in range(5):
        for r in (rot1 if i % 2 == 0 else rot2):
            x0 = (x0 + x1).astype(np.uint32)
            x1 = rotl(x1, r)
            x1 = (x1 ^ x0).astype(np.uint32)
        x0 = (x0 + ks[(i + 1) % 3]).astype(np.uint32)
        x1 = (x1 + ks[(i + 2) % 3] + np.uint32(i + 1)).astype(np.uint32)
    bits = (x0 ^ x1).astype(np.uint32)
    f = ((bits >> np.uint32(9)) | np.uint32(0x3F800000)).view(np.float32)
    u = np.abs(np.maximum(np.float32(0.0), f - np.float32(1.0)))
    eps = np.float32(1e-10)
    g = (-np.log(eps - np.log(u + eps))).astype(np.float32)
    return g.reshape(_B, _V)


_GUMBEL = _make_gumbel()


def _argmax_body(x_ref, g_ref, idx_ref, vmax_ref):
    j = pl.program_id(0)
    x = x_ref[...] + g_ref[...]
    col = jax.lax.broadcasted_iota(jnp.int32, (_B, _BVA), 1) + j * _BVA
    x = jnp.where(col < _V, x, -jnp.inf)
    bm = jnp.max(x, axis=1, keepdims=True)
    bidx = jnp.min(jnp.where(x == bm, col, jnp.int32(2**31 - 1)),
                   axis=1, keepdims=True)

    @pl.when(j == 0)
    def _():
        vmax_ref[...] = bm
        idx_ref[...] = bidx

    @pl.when(j > 0)
    def _():
        upd = bm > vmax_ref[...]
        vmax_ref[...] = jnp.where(upd, bm, vmax_ref[...])
        idx_ref[...] = jnp.where(upd, bidx, idx_ref[...])


def _onehot_body(idx_ref, o_ref):
    j = pl.program_id(0)
    col = jax.lax.broadcasted_iota(jnp.int32, (_B, _BVO), 1) + j * _BVO
    o_ref[...] = (col == idx_ref[...]).astype(jnp.float32)


@jax.jit
def kernel(logits):
    idx = pl.pallas_call(
        _argmax_body,
        grid=(_NBA,),
        in_specs=[pl.BlockSpec((_B, _BVA), lambda j: (0, j)),
                  pl.BlockSpec((_B, _BVA), lambda j: (0, j))],
        out_specs=pl.BlockSpec((_B, 1), lambda j: (0, 0)),
        out_shape=jax.ShapeDtypeStruct((_B, 1), jnp.int32),
        scratch_shapes=[pltpu.VMEM((_B, 1), jnp.float32)],
    )(logits, _GUMBEL)
    out = pl.pallas_call(
        _onehot_body,
        grid=(_NBO,),
        in_specs=[pl.BlockSpec((_B, 1), lambda j: (0, 0))],
        out_specs=pl.BlockSpec((_B, _BVO), lambda j: (0, j)),
        out_shape=jax.ShapeDtypeStruct((_B, _V), jnp.float32),
    )(idx)
    return out
